# Initial kernel scaffold; baseline (speedup 1.0000x reference)
import jax
import jax.numpy as jnp
from jax.experimental import pallas as pl


def kernel(x, edge_index, W, bias):
    raise NotImplementedError("write your pallas kernel here")


# speedup vs baseline: 5.4979x; 5.4979x over previous
import functools

import jax
import jax.numpy as jnp
from jax import lax
from jax.experimental import pallas as pl
from jax.experimental.pallas import tpu as pltpu
from jax.experimental.pallas import tpu_sc as plsc

N_NODES = 10000
N_PAD = 10240
N_EDGES = 320000
D = 128

NC = 2
NS = 16
NW = NC * NS
EPW = N_EDGES // NW
CH = 80
NCH = EPW // CH
ROWS_PER_TILE = N_PAD // NS


def _sc_aggregate(x, src, dst, zeros):
    mesh = plsc.VectorSubcoreMesh(core_axis_name="c", subcore_axis_name="s")

    @functools.partial(
        pl.kernel,
        mesh=mesh,
        out_type=jax.ShapeDtypeStruct((NC, N_PAD, D), jnp.float32),
        scratch_types=[
            pltpu.VMEM((CH,), jnp.int32),
            pltpu.VMEM((CH,), jnp.int32),
            pltpu.VMEM((CH, D), jnp.float32),
            pltpu.VMEM_SHARED((N_PAD, D), jnp.float32),
            pltpu.SemaphoreType.DMA,
        ],
    )
    def agg(x_hbm, src_hbm, dst_hbm, zeros_hbm, out_hbm, sidx, didx, rows, acc, sem):
        cid = lax.axis_index("c")
        sid = lax.axis_index("s")
        wid = sid * NC + cid

        rbase = sid * ROWS_PER_TILE
        pltpu.sync_copy(zeros_hbm.at[pl.ds(rbase, ROWS_PER_TILE)],
                        acc.at[pl.ds(rbase, ROWS_PER_TILE)])
        plsc.subcore_barrier()

        ebase = wid * EPW

        def body(j, carry):
            off = ebase + j * CH
            pltpu.sync_copy(src_hbm.at[pl.ds(off, CH)], sidx)
            pltpu.sync_copy(dst_hbm.at[pl.ds(off, CH)], didx)
            pltpu.async_copy(x_hbm.at[sidx], rows, sem).wait()
            pltpu.sync_copy(rows, acc.at[didx], add=True)
            return carry

        lax.fori_loop(0, NCH, body, 0)
        plsc.subcore_barrier()

        pltpu.sync_copy(acc.at[pl.ds(rbase, ROWS_PER_TILE)],
                        out_hbm.at[cid, pl.ds(rbase, ROWS_PER_TILE)])

    return agg(x, src, dst, zeros)


def _tc_combine(partials, W, bias):
    BR = 1000

    def body(p_ref, w_ref, b_ref, o_ref):
        s = p_ref[0] + p_ref[1]
        o_ref[...] = lax.dot_general(
            s, w_ref[...], (((1,), (1,)), ((), ())),
            preferred_element_type=jnp.float32) + b_ref[...]

    return pl.pallas_call(
        body,
        grid=(N_NODES // BR,),
        in_specs=[
            pl.BlockSpec((NC, BR, D), lambda i: (0, i, 0)),
            pl.BlockSpec((D, D), lambda i: (0, 0)),
            pl.BlockSpec((1, D), lambda i: (0, 0)),
        ],
        out_specs=pl.BlockSpec((BR, D), lambda i: (i, 0)),
        out_shape=jax.ShapeDtypeStruct((N_NODES, D), jnp.float32),
    )(partials, W, bias.reshape(1, D))


def kernel(x, edge_index, W, bias):
    src = edge_index[0].astype(jnp.int32)
    dst = edge_index[1].astype(jnp.int32)
    zeros = jnp.zeros((N_PAD, D), jnp.float32)
    partials = _sc_aggregate(x, src, dst, zeros)
    return _tc_combine(partials, W, bias)


# speedup vs baseline: 12.1778x; 2.2150x over previous
import functools

import jax
import jax.numpy as jnp
from jax import lax
from jax.experimental import pallas as pl
from jax.experimental.pallas import tpu as pltpu
from jax.experimental.pallas import tpu_sc as plsc

N_NODES = 10000
N_PAD = 10240
N_EDGES = 320000
D = 128

NC = 2
NS = 16
NW = NC * NS
EPW = N_EDGES // NW
CH = 125
NCH = EPW // CH
PH = NCH // 2
ROWS_PER_TILE = N_PAD // NS


def _sc_aggregate(x, src, dst, zeros):
    mesh = plsc.VectorSubcoreMesh(core_axis_name="c", subcore_axis_name="s")

    @functools.partial(
        pl.kernel,
        mesh=mesh,
        out_type=jax.ShapeDtypeStruct((NC, N_PAD, D), jnp.float32),
        scratch_types=[
            pltpu.VMEM((PH, CH), jnp.int32),
            pltpu.VMEM((PH, CH), jnp.int32),
            pltpu.VMEM((CH, D), jnp.float32),
            pltpu.VMEM((CH, D), jnp.float32),
            pltpu.VMEM_SHARED((N_PAD, D), jnp.float32),
            pltpu.SemaphoreType.DMA,
            pltpu.SemaphoreType.DMA,
        ],
    )
    def agg(x_hbm, src_hbm, dst_hbm, zeros_hbm, out_hbm,
            sidx, didx, rows_a, rows_b, acc, gsem_a, gsem_b):
        cid = lax.axis_index("c")
        sid = lax.axis_index("s")
        wid = sid * NC + cid

        rbase = sid * ROWS_PER_TILE
        pltpu.sync_copy(zeros_hbm.at[pl.ds(rbase, ROWS_PER_TILE)],
                        acc.at[pl.ds(rbase, ROWS_PER_TILE)])
        plsc.subcore_barrier()

        for p in range(NCH // PH):
            pltpu.sync_copy(src_hbm.at[wid, pl.ds(p * PH, PH)], sidx)
            pltpu.sync_copy(dst_hbm.at[wid, pl.ds(p * PH, PH)], didx)
            pltpu.async_copy(x_hbm.at[sidx.at[0]], rows_a, gsem_a)

            def body(j2, carry):
                a = 2 * j2
                pltpu.async_copy(x_hbm.at[sidx.at[a + 1]], rows_b, gsem_b)
                pltpu.make_async_copy(x_hbm.at[sidx.at[a]], rows_a, gsem_a).wait()
                pltpu.sync_copy(rows_a, acc.at[didx.at[a]], add=True)

                @pl.when(j2 < PH // 2 - 1)
                def _():
                    pltpu.async_copy(x_hbm.at[sidx.at[a + 2]], rows_a, gsem_a)

                pltpu.make_async_copy(x_hbm.at[sidx.at[a + 1]], rows_b, gsem_b).wait()
                pltpu.sync_copy(rows_b, acc.at[didx.at[a + 1]], add=True)
                return carry

            lax.fori_loop(0, PH // 2, body, 0)
        plsc.subcore_barrier()

        pltpu.sync_copy(acc.at[pl.ds(rbase, ROWS_PER_TILE)],
                        out_hbm.at[cid, pl.ds(rbase, ROWS_PER_TILE)])

    return agg(x, src, dst, zeros)


def _tc_combine(partials, W, bias):
    BR = 1000

    def body(p_ref, w_ref, b_ref, o_ref):
        s = p_ref[0] + p_ref[1]
        o_ref[...] = lax.dot_general(
            s, w_ref[...], (((1,), (1,)), ((), ())),
            preferred_element_type=jnp.float32) + b_ref[...]

    return pl.pallas_call(
        body,
        grid=(N_NODES // BR,),
        in_specs=[
            pl.BlockSpec((NC, BR, D), lambda i: (0, i, 0)),
            pl.BlockSpec((D, D), lambda i: (0, 0)),
            pl.BlockSpec((1, D), lambda i: (0, 0)),
        ],
        out_specs=pl.BlockSpec((BR, D), lambda i: (i, 0)),
        out_shape=jax.ShapeDtypeStruct((N_NODES, D), jnp.float32),
    )(partials, W, bias.reshape(1, D))


def kernel(x, edge_index, W, bias):
    src = edge_index[0].astype(jnp.int32).reshape(NW, NCH, CH)
    dst = edge_index[1].astype(jnp.int32).reshape(NW, NCH, CH)
    zeros = jnp.zeros((N_PAD, D), jnp.float32)
    partials = _sc_aggregate(x, src, dst, zeros)
    return _tc_combine(partials, W, bias)


# speedup vs baseline: 12.3827x; 1.0168x over previous
import functools

import jax
import jax.numpy as jnp
from jax import lax
from jax.experimental import pallas as pl
from jax.experimental.pallas import tpu as pltpu
from jax.experimental.pallas import tpu_sc as plsc

N_NODES = 10000
N_PAD = 10240
N_EDGES = 320000
D = 128

NC = 2
NS = 16
NW = NC * NS
EPW = N_EDGES // NW
CH = 125
NCH = EPW // CH
NPH = 4
PH = NCH // NPH
ROWS_PER_TILE = N_PAD // NS


def _sc_aggregate(x, src, dst, zeros):
    mesh = plsc.VectorSubcoreMesh(core_axis_name="c", subcore_axis_name="s")

    @functools.partial(
        pl.kernel,
        mesh=mesh,
        out_type=jax.ShapeDtypeStruct((NC, N_PAD, D), jnp.float32),
        scratch_types=[
            pltpu.VMEM((PH, CH), jnp.int32),
            pltpu.VMEM((PH, CH), jnp.int32),
            pltpu.VMEM((PH, CH), jnp.int32),
            pltpu.VMEM((PH, CH), jnp.int32),
            pltpu.VMEM((CH, D), jnp.float32),
            pltpu.VMEM((CH, D), jnp.float32),
            pltpu.VMEM_SHARED((N_PAD, D), jnp.float32),
            pltpu.SemaphoreType.DMA,
            pltpu.SemaphoreType.DMA,
            pltpu.SemaphoreType.DMA,
            pltpu.SemaphoreType.DMA,
            pltpu.SemaphoreType.DMA,
        ],
    )
    def agg(x_hbm, src_hbm, dst_hbm, zeros_hbm, out_hbm,
            sidx0, didx0, sidx1, didx1, rows_a, rows_b, acc,
            gsem_a, gsem_b, zsem, isem0, isem1):
        cid = lax.axis_index("c")
        sid = lax.axis_index("s")
        wid = sid * NC + cid

        rbase = sid * ROWS_PER_TILE
        zcopy = pltpu.async_copy(zeros_hbm.at[pl.ds(rbase, ROWS_PER_TILE)],
                                 acc.at[pl.ds(rbase, ROWS_PER_TILE)], zsem)
        idx_bufs = [(sidx0, didx0, isem0), (sidx1, didx1, isem1)]
        pltpu.sync_copy(src_hbm.at[wid, 0], sidx0)
        pltpu.sync_copy(dst_hbm.at[wid, 0], didx0)
        pending = {1: (pltpu.async_copy(src_hbm.at[wid, 1], sidx1, isem1),
                       pltpu.async_copy(dst_hbm.at[wid, 1], didx1, isem1))}
        pltpu.async_copy(x_hbm.at[sidx0.at[0]], rows_a, gsem_a)
        zcopy.wait()
        plsc.subcore_barrier()

        for p in range(NPH):
            sidx, didx, _ = idx_bufs[p % 2]
            if 1 <= p and p + 1 < NPH:
                ns, nd, nsem = idx_bufs[(p + 1) % 2]
                pending[p + 1] = (
                    pltpu.async_copy(src_hbm.at[wid, p + 1], ns, nsem),
                    pltpu.async_copy(dst_hbm.at[wid, p + 1], nd, nsem))
            for h in pending.pop(p, ()):
                h.wait()
            if p > 0:
                pltpu.async_copy(x_hbm.at[sidx.at[0]], rows_a, gsem_a)

            def body(j2, carry, sidx=sidx, didx=didx):
                a = 2 * j2
                pltpu.async_copy(x_hbm.at[sidx.at[a + 1]], rows_b, gsem_b)
                pltpu.make_async_copy(x_hbm.at[sidx.at[a]], rows_a, gsem_a).wait()
                pltpu.sync_copy(rows_a, acc.at[didx.at[a]], add=True)

                @pl.when(j2 < PH // 2 - 1)
                def _():
                    pltpu.async_copy(x_hbm.at[sidx.at[a + 2]], rows_a, gsem_a)

                pltpu.make_async_copy(x_hbm.at[sidx.at[a + 1]], rows_b, gsem_b).wait()
                pltpu.sync_copy(rows_b, acc.at[didx.at[a + 1]], add=True)
                return carry

            lax.fori_loop(0, PH // 2, body, 0)
        plsc.subcore_barrier()

        pltpu.sync_copy(acc.at[pl.ds(rbase, ROWS_PER_TILE)],
                        out_hbm.at[cid, pl.ds(rbase, ROWS_PER_TILE)])

    return agg(x, src, dst, zeros)


def _tc_combine(partials, W, bias):
    BR = 1000

    def body(p_ref, w_ref, b_ref, o_ref):
        s = p_ref[0] + p_ref[1]
        o_ref[...] = lax.dot_general(
            s, w_ref[...], (((1,), (1,)), ((), ())),
            preferred_element_type=jnp.float32) + b_ref[...]

    return pl.pallas_call(
        body,
        grid=(N_NODES // BR,),
        in_specs=[
            pl.BlockSpec((NC, BR, D), lambda i: (0, i, 0)),
            pl.BlockSpec((D, D), lambda i: (0, 0)),
            pl.BlockSpec((1, D), lambda i: (0, 0)),
        ],
        out_specs=pl.BlockSpec((BR, D), lambda i: (i, 0)),
        out_shape=jax.ShapeDtypeStruct((N_NODES, D), jnp.float32),
    )(partials, W, bias.reshape(1, D))


def kernel(x, edge_index, W, bias):
    src = edge_index[0].astype(jnp.int32).reshape(NW, NPH, PH, CH)
    dst = edge_index[1].astype(jnp.int32).reshape(NW, NPH, PH, CH)
    zeros = jnp.zeros((N_PAD, D), jnp.float32)
    partials = _sc_aggregate(x, src, dst, zeros)
    return _tc_combine(partials, W, bias)
